# baseline (speedup 1.0000x reference)
import jax
import jax.numpy as jnp
from jax import lax
from jax.experimental import pallas as pl
from jax.experimental.pallas import tpu as pltpu
from jax.experimental.pallas import tpu_sc as plsc

EMBED = 16
BIASN = 5
BATCH = 16384
VOCAB = 1000000
NCORES = 2
NSUBCORES = 16
NWORKERS = NCORES * NSUBCORES
BPW = BATCH // NWORKERS
BLK = 32768
NBLK = (VOCAB + BLK - 1) // BLK


def _em_body(src_ref, dst_ref, *plane_refs):
    x = src_ref[...]
    dst_ref[...] = x
    for d in range(EMBED):
        plane_refs[d][...] = jnp.reshape(x[d:d + 1, :], (BLK,))


def _em_pass(em_t):
    outs = pl.pallas_call(
        _em_body,
        grid=(NBLK,),
        in_specs=[pl.BlockSpec((EMBED, BLK), lambda g: (0, g))],
        out_specs=[pl.BlockSpec((EMBED, BLK), lambda g: (0, g))] + [
            pl.BlockSpec((BLK,), lambda g: (g,)) for _ in range(EMBED)],
        out_shape=[jax.ShapeDtypeStruct((EMBED, VOCAB), jnp.float32)] + [
            jax.ShapeDtypeStruct((VOCAB,), jnp.float32)
            for _ in range(EMBED)],
        compiler_params=pltpu.CompilerParams(
            dimension_semantics=("arbitrary",)),
    )(em_t)
    return outs[0], outs[1:]


CW = 7808
CPW = 31232
NCHUNK = CPW // CW
TAIL = VOCAB - CPW * NWORKERS


def _sc_bias_copy_body(bs_in, bs_out, buf, tailbuf, sem):
    wid = lax.axis_index("s") * NCORES + lax.axis_index("c")
    c0 = pl.multiple_of(wid * CPW, 128)
    for k in range(NCHUNK):
        pltpu.sync_copy(bs_in.at[:, pl.ds(c0 + k * CW, CW)], buf)
        pltpu.sync_copy(buf, bs_out.at[:, pl.ds(c0 + k * CW, CW)])

    @pl.when(wid == 0)
    def _tail():
        t0 = CPW * NWORKERS
        pltpu.sync_copy(bs_in.at[:, pl.ds(t0, TAIL)], tailbuf)
        pltpu.sync_copy(tailbuf, bs_out.at[:, pl.ds(t0, TAIL)])


def _sc_bias_copy(bs_t):
    mesh = plsc.VectorSubcoreMesh(core_axis_name="c", subcore_axis_name="s")
    k = pl.kernel(
        _sc_bias_copy_body,
        out_type=jax.ShapeDtypeStruct((BIASN, VOCAB), jnp.float32),
        mesh=mesh,
        scratch_types=[
            pltpu.VMEM((BIASN, CW), jnp.float32),
            pltpu.VMEM((BIASN, TAIL), jnp.float32),
            pltpu.SemaphoreType.DMA,
        ],
    )
    return k(bs_t)


def _sc_gather_body(*refs):
    planes = refs[:EMBED]
    idx_hbm = refs[EMBED]
    outs = refs[EMBED + 1:EMBED + 1 + EMBED]
    idx_v = refs[EMBED + 1 + EMBED]
    rvs = refs[EMBED + 2 + EMBED:EMBED + 2 + 2 * EMBED]
    sem = refs[-1]

    wid = lax.axis_index("s") * NCORES + lax.axis_index("c")
    base = wid * BPW
    pltpu.sync_copy(idx_hbm.at[pl.ds(base, BPW)], idx_v)
    handles = [pltpu.async_copy(planes[d].at[idx_v], rvs[d], sem)
               for d in range(EMBED)]
    for h in handles:
        h.wait()
    for d in range(EMBED):
        pltpu.sync_copy(rvs[d], outs[d].at[0, pl.ds(base, BPW)])


def _sc_gather(planes, idx):
    mesh = plsc.VectorSubcoreMesh(core_axis_name="c", subcore_axis_name="s")
    k = pl.kernel(
        _sc_gather_body,
        out_type=[jax.ShapeDtypeStruct((1, BATCH), jnp.float32)
                  for _ in range(EMBED)],
        mesh=mesh,
        scratch_types=[pltpu.VMEM((BPW,), jnp.int32)] + [
            pltpu.VMEM((BPW,), jnp.float32) for _ in range(EMBED)] + [
            pltpu.SemaphoreType.DMA],
        compiler_params=pltpu.CompilerParams(use_tc_tiling_on_sc=False),
    )
    return k(*planes, idx)


def _fused_body(ctx_sref, *refs):
    rows_refs = refs[:EMBED]
    labels_ref = refs[EMBED]
    em_blk = refs[EMBED + 1]
    bs_blk = refs[EMBED + 2]
    acc_ref = refs[EMBED + 3]
    em_out = refs[EMBED + 4]
    bs_out = refs[EMBED + 5]

    off = ctx_sref[0] % 128
    onehot = (lax.broadcasted_iota(jnp.int32, (128, 1), 0) == off
              ).astype(jnp.float32)
    ctx16 = lax.dot_general(em_blk[...], onehot, (((1,), (0,)), ((), ())),
                            preferred_element_type=jnp.float32)
    bctx = lax.dot_general(bs_blk[...], onehot, (((1,), (0,)), ((), ())),
                           preferred_element_type=jnp.float32)

    rows = jnp.concatenate([r[...] for r in rows_refs], axis=0)
    dots = lax.dot_general(ctx16, rows, (((0,), (0,)), ((), ())),
                           preferred_element_type=jnp.float32)
    labels = labels_ref[...]
    labels_f = labels.astype(jnp.float32)

    db = dots - bctx
    iota5 = lax.broadcasted_iota(jnp.int32, (BIASN, 1), 0)

    firstz = jnp.min(jnp.where(db <= 0, iota5, BIASN), axis=0, keepdims=True)
    p = jnp.where(firstz < BIASN, firstz + 1, BIASN + 1)
    acc_ref[0, 0] = jnp.sum((p == labels).astype(jnp.float32)) / BATCH

    ytp = iota5 < labels
    judge_pos = ((db > 0) & ytp) | ((db < 0) & (~ytp))
    tau = jnp.where(judge_pos, 0.0, labels_f)

    bu = jnp.sum(tau, axis=1, keepdims=True) / BATCH
    ts = jnp.sum(tau, axis=0, keepdims=True)
    wu = lax.dot_general(rows, ts, (((1,), (1,)), ((), ())),
                         preferred_element_type=jnp.float32) / BATCH

    m = (lax.broadcasted_iota(jnp.int32, (1, 128), 1) == off
         ).astype(jnp.float32)
    em_out[...] = em_blk[...] + wu * m
    bs_out[...] = bs_blk[...] - bu * m


def _fused(context_id, rows16, labels, em_copy, bs_t):
    n_in = EMBED + 4
    return pl.pallas_call(
        _fused_body,
        grid_spec=pltpu.PrefetchScalarGridSpec(
            num_scalar_prefetch=1,
            grid=(1,),
            in_specs=[pl.BlockSpec(memory_space=pltpu.VMEM)
                      for _ in range(EMBED + 1)] + [
                pl.BlockSpec((EMBED, 128), lambda g, c: (0, c[0] // 128)),
                pl.BlockSpec((BIASN, 128), lambda g, c: (0, c[0] // 128)),
            ],
            out_specs=[
                pl.BlockSpec(memory_space=pltpu.SMEM),
                pl.BlockSpec((EMBED, 128), lambda g, c: (0, c[0] // 128)),
                pl.BlockSpec((BIASN, 128), lambda g, c: (0, c[0] // 128)),
            ],
        ),
        out_shape=[
            jax.ShapeDtypeStruct((1, 1), jnp.float32),
            jax.ShapeDtypeStruct((EMBED, VOCAB), jnp.float32),
            jax.ShapeDtypeStruct((BIASN, VOCAB), jnp.float32),
        ],
        input_output_aliases={n_in - 2: 1, n_in - 1: 2},
    )(context_id, *rows16, labels, em_copy, bs_t)


def kernel(in_embed, in_bias, context_id, target_ids, labels):
    em_t = in_embed.T
    bs_t = in_bias.T
    tgt = target_ids.reshape(-1)
    bs_copy = _sc_bias_copy(bs_t)
    em_copy, planes = _em_pass(em_t)
    rows16 = _sc_gather(planes, tgt)
    acc, em2, bs2 = _fused(context_id, rows16, labels.reshape(1, BATCH),
                           em_copy, bs_copy)
    return acc.reshape(()), em2.T, bs2.T


# speedup vs baseline: 1.0335x; 1.0335x over previous
import jax
import jax.numpy as jnp
from jax import lax
from jax.experimental import pallas as pl
from jax.experimental.pallas import tpu as pltpu
from jax.experimental.pallas import tpu_sc as plsc

EMBED = 16
BIASN = 5
BATCH = 16384
VOCAB = 1000000
NCORES = 2
NSUBCORES = 16
NWORKERS = NCORES * NSUBCORES
BPW = BATCH // NWORKERS
BLK = 65536
NBLK = (VOCAB + BLK - 1) // BLK


def _em_body(src_ref, dst_ref, *plane_refs):
    x = src_ref[...]
    dst_ref[...] = x
    for d in range(EMBED):
        plane_refs[d][...] = jnp.reshape(x[d:d + 1, :], (BLK,))


def _em_pass(em_t):
    outs = pl.pallas_call(
        _em_body,
        grid=(NBLK,),
        in_specs=[pl.BlockSpec((EMBED, BLK), lambda g: (0, g))],
        out_specs=[pl.BlockSpec((EMBED, BLK), lambda g: (0, g))] + [
            pl.BlockSpec((BLK,), lambda g: (g,)) for _ in range(EMBED)],
        out_shape=[jax.ShapeDtypeStruct((EMBED, VOCAB), jnp.float32)] + [
            jax.ShapeDtypeStruct((VOCAB,), jnp.float32)
            for _ in range(EMBED)],
        compiler_params=pltpu.CompilerParams(
            dimension_semantics=("arbitrary",)),
    )(em_t)
    return outs[0], outs[1:]


def _sc_gather_body(*refs):
    planes = refs[:EMBED]
    idx_hbm = refs[EMBED]
    outs = refs[EMBED + 1:EMBED + 1 + EMBED]
    idx_v = refs[EMBED + 1 + EMBED]
    rvs = refs[EMBED + 2 + EMBED:EMBED + 2 + 2 * EMBED]
    sem = refs[-1]

    wid = lax.axis_index("s") * NCORES + lax.axis_index("c")
    base = wid * BPW
    pltpu.sync_copy(idx_hbm.at[pl.ds(base, BPW)], idx_v)
    handles = [pltpu.async_copy(planes[d].at[idx_v], rvs[d], sem)
               for d in range(EMBED)]
    for h in handles:
        h.wait()
    for d in range(EMBED):
        pltpu.sync_copy(rvs[d], outs[d].at[0, pl.ds(base, BPW)])


def _sc_gather(planes, idx):
    mesh = plsc.VectorSubcoreMesh(core_axis_name="c", subcore_axis_name="s")
    k = pl.kernel(
        _sc_gather_body,
        out_type=[jax.ShapeDtypeStruct((1, BATCH), jnp.float32)
                  for _ in range(EMBED)],
        mesh=mesh,
        scratch_types=[pltpu.VMEM((BPW,), jnp.int32)] + [
            pltpu.VMEM((BPW,), jnp.float32) for _ in range(EMBED)] + [
            pltpu.SemaphoreType.DMA],
        compiler_params=pltpu.CompilerParams(use_tc_tiling_on_sc=False),
    )
    return k(*planes, idx)


def _fused_body(ctx_sref, *refs):
    rows_refs = refs[:EMBED]
    labels_ref = refs[EMBED]
    em_blk = refs[EMBED + 1]
    bs_blk = refs[EMBED + 2]
    acc_ref = refs[EMBED + 3]
    em_out = refs[EMBED + 4]
    bs_out = refs[EMBED + 5]

    off = ctx_sref[0] % 128
    onehot = (lax.broadcasted_iota(jnp.int32, (128, 1), 0) == off
              ).astype(jnp.float32)
    ctx16 = lax.dot_general(em_blk[...], onehot, (((1,), (0,)), ((), ())),
                            preferred_element_type=jnp.float32)
    bctx = lax.dot_general(bs_blk[...], onehot, (((1,), (0,)), ((), ())),
                           preferred_element_type=jnp.float32)

    rows = jnp.concatenate([r[...] for r in rows_refs], axis=0)
    dots = lax.dot_general(ctx16, rows, (((0,), (0,)), ((), ())),
                           preferred_element_type=jnp.float32)
    labels = labels_ref[...]
    labels_f = labels.astype(jnp.float32)

    db = dots - bctx
    iota5 = lax.broadcasted_iota(jnp.int32, (BIASN, 1), 0)

    firstz = jnp.min(jnp.where(db <= 0, iota5, BIASN), axis=0, keepdims=True)
    p = jnp.where(firstz < BIASN, firstz + 1, BIASN + 1)
    acc_ref[0, 0] = jnp.sum((p == labels).astype(jnp.float32)) / BATCH

    ytp = iota5 < labels
    judge_pos = ((db > 0) & ytp) | ((db < 0) & (~ytp))
    tau = jnp.where(judge_pos, 0.0, labels_f)

    bu = jnp.sum(tau, axis=1, keepdims=True) / BATCH
    ts = jnp.sum(tau, axis=0, keepdims=True)
    wu = lax.dot_general(rows, ts, (((1,), (1,)), ((), ())),
                         preferred_element_type=jnp.float32) / BATCH

    m = (lax.broadcasted_iota(jnp.int32, (1, 128), 1) == off
         ).astype(jnp.float32)
    em_out[...] = em_blk[...] + wu * m
    bs_out[...] = bs_blk[...] - bu * m


def _fused(context_id, rows16, labels, em_copy, bs_t):
    n_in = EMBED + 4
    return pl.pallas_call(
        _fused_body,
        grid_spec=pltpu.PrefetchScalarGridSpec(
            num_scalar_prefetch=1,
            grid=(1,),
            in_specs=[pl.BlockSpec(memory_space=pltpu.VMEM)
                      for _ in range(EMBED + 1)] + [
                pl.BlockSpec((EMBED, 128), lambda g, c: (0, c[0] // 128)),
                pl.BlockSpec((BIASN, 128), lambda g, c: (0, c[0] // 128)),
            ],
            out_specs=[
                pl.BlockSpec(memory_space=pltpu.SMEM),
                pl.BlockSpec((EMBED, 128), lambda g, c: (0, c[0] // 128)),
                pl.BlockSpec((BIASN, 128), lambda g, c: (0, c[0] // 128)),
            ],
        ),
        out_shape=[
            jax.ShapeDtypeStruct((1, 1), jnp.float32),
            jax.ShapeDtypeStruct((EMBED, VOCAB), jnp.float32),
            jax.ShapeDtypeStruct((BIASN, VOCAB), jnp.float32),
        ],
        input_output_aliases={n_in - 2: 1, n_in - 1: 2},
    )(context_id, *rows16, labels, em_copy, bs_t)


def kernel(in_embed, in_bias, context_id, target_ids, labels):
    em_t = in_embed.T
    bs_t = in_bias.T
    tgt = target_ids.reshape(-1)
    em_copy, planes = _em_pass(em_t)
    rows16 = _sc_gather(planes, tgt)
    acc, em2, bs2 = _fused(context_id, rows16, labels.reshape(1, BATCH),
                           em_copy, bs_t)
    return acc.reshape(()), em2.T, bs2.T
